# Initial kernel scaffold; baseline (speedup 1.0000x reference)
import jax
import jax.numpy as jnp
from jax.experimental import pallas as pl


def kernel(x, edge_index, Wl0, bl0, Wr0, Wl1, bl1, Wr1, Wl2, bl2, Wr2, Wlin0, blin0, Wlin1, blin1, Wout, bout):
    raise NotImplementedError("write your pallas kernel here")


# speedup vs baseline: 6.6102x; 6.6102x over previous
import functools

import jax
import jax.numpy as jnp
from jax import lax
from jax.experimental import pallas as pl
from jax.experimental.pallas import tpu as pltpu
from jax.experimental.pallas import tpu_sc as plsc

N = 10000
E = 320000
D_IN = 128
H = 64
D_OUT = 128

NC = 2
NS = 16
NW = NC * NS
CHUNK = 128
CH = 79
EPAD = NW * CH * CHUNK
NTAB = 10240
ROWS_PER_TILE = NTAB // NS
W0 = 80


def _elu(v):
    return jnp.where(v > 0, v, jnp.exp(jnp.minimum(v, 0.0)) - 1.0)


def _make_sc_segsum(W):
    mesh = plsc.VectorSubcoreMesh(core_axis_name="c", subcore_axis_name="s")

    @functools.partial(
        pl.kernel,
        out_type=jax.ShapeDtypeStruct((NC, NTAB, W), jnp.float32),
        mesh=mesh,
        scratch_types=[
            pltpu.VMEM((CH, CHUNK), jnp.int32),
            pltpu.VMEM((CH, CHUNK), jnp.int32),
            pltpu.VMEM((CHUNK, W), jnp.float32),
            pltpu.VMEM_SHARED((NTAB, W), jnp.float32),
            pltpu.SemaphoreType.DMA,
        ],
        compiler_params=pltpu.CompilerParams(use_tc_tiling_on_sc=False),
    )
    def sc_segsum(tab, srcs, dsts, zeros, out, src_v, dst_v, buf, agg_sh, sem):
        cid = lax.axis_index("c")
        sid = lax.axis_index("s")
        wid = sid * NC + cid

        pltpu.sync_copy(zeros, buf)

        @pl.loop(0, ROWS_PER_TILE // CHUNK)
        def _zero(k):
            r = sid * ROWS_PER_TILE + k * CHUNK
            pltpu.sync_copy(buf, agg_sh.at[pl.ds(r, CHUNK)])

        pltpu.sync_copy(srcs.at[wid], src_v)
        pltpu.sync_copy(dsts.at[wid], dst_v)
        plsc.subcore_barrier()

        @pl.loop(0, CH)
        def _edges(j):
            pltpu.async_copy(tab.at[src_v.at[j]], buf, sem).wait()
            pltpu.sync_copy(buf, agg_sh.at[dst_v.at[j]], add=True)

        plsc.subcore_barrier()

        @pl.loop(0, ROWS_PER_TILE // CHUNK)
        def _dump(k):
            r = sid * ROWS_PER_TILE + k * CHUNK
            pltpu.sync_copy(agg_sh.at[pl.ds(r, CHUNK)], buf)
            pltpu.sync_copy(buf, out.at[cid, pl.ds(r, CHUNK)])

    return sc_segsum


_sc_segsum_80 = _make_sc_segsum(W0)
_sc_segsum_64 = _make_sc_segsum(H)


def _tc1_body(x_ref, w_ref, out_ref):
    p = jnp.dot(x_ref[...], w_ref[...], preferred_element_type=jnp.float32)
    col = lax.broadcasted_iota(jnp.int32, (N, W0), 1)
    out_ref[...] = p + (col == H).astype(jnp.float32)


def _tc2_body(agg_ref, x_ref, wr_ref, bl_ref, wl_ref, p1_ref, h1_ref, rc_ref):
    a = agg_ref[0, :N, :] + agg_ref[1, :N, :]
    cnt = a[:, H:H + 1]
    rc = 1.0 / jnp.maximum(cnt, 1.0)
    mean = a[:, :H] * rc
    s = jnp.dot(x_ref[...], wr_ref[...], preferred_element_type=jnp.float32)
    h1 = _elu(mean + bl_ref[...] + s)
    h1_ref[...] = h1
    p1_ref[...] = jnp.dot(h1, wl_ref[...], preferred_element_type=jnp.float32)
    rc_ref[...] = rc


def _tc3_body(agg_ref, h_ref, wr_ref, bl_ref, wl_ref, rc_ref, p2_ref, h2_ref):
    a = agg_ref[0, :N, :] + agg_ref[1, :N, :]
    mean = a * rc_ref[...]
    s = jnp.dot(h_ref[...], wr_ref[...], preferred_element_type=jnp.float32)
    h2 = _elu(mean + bl_ref[...] + s)
    h2_ref[...] = h2
    p2_ref[...] = jnp.dot(h2, wl_ref[...], preferred_element_type=jnp.float32)


def _tc4_body(agg_ref, h_ref, wr_ref, bl_ref, rc_ref, w0_ref, b0_ref, w1_ref,
              b1_ref, wo_ref, bo_ref, out_ref):
    a = agg_ref[0, :N, :] + agg_ref[1, :N, :]
    mean = a * rc_ref[...]
    s = jnp.dot(h_ref[...], wr_ref[...], preferred_element_type=jnp.float32)
    h3 = _elu(mean + bl_ref[...] + s)
    t = _elu(jnp.dot(h3, w0_ref[...], preferred_element_type=jnp.float32) + b0_ref[...])
    t = _elu(jnp.dot(t, w1_ref[...], preferred_element_type=jnp.float32) + b1_ref[...])
    out_ref[...] = jnp.dot(t, wo_ref[...], preferred_element_type=jnp.float32) + bo_ref[...]


_f32 = jnp.float32

_tc1 = pl.pallas_call(_tc1_body, out_shape=jax.ShapeDtypeStruct((N, W0), _f32))
_tc2 = pl.pallas_call(
    _tc2_body,
    out_shape=(
        jax.ShapeDtypeStruct((N, H), _f32),
        jax.ShapeDtypeStruct((N, H), _f32),
        jax.ShapeDtypeStruct((N, 1), _f32),
    ),
)
_tc3 = pl.pallas_call(
    _tc3_body,
    out_shape=(
        jax.ShapeDtypeStruct((N, H), _f32),
        jax.ShapeDtypeStruct((N, H), _f32),
    ),
)
_tc4 = pl.pallas_call(_tc4_body, out_shape=jax.ShapeDtypeStruct((N, D_OUT), _f32))


def kernel(x, edge_index, Wl0, bl0, Wr0, Wl1, bl1, Wr1, Wl2, bl2, Wr2,
           Wlin0, blin0, Wlin1, blin1, Wout, bout):
    src = edge_index[0]
    dst = edge_index[1]
    pad = EPAD - E
    src_p = jnp.concatenate([src, jnp.zeros((pad,), jnp.int32)]).reshape(NW, CH, CHUNK)
    dst_p = jnp.concatenate([dst, jnp.full((pad,), N, jnp.int32)]).reshape(NW, CH, CHUNK)
    zeros80 = jnp.zeros((CHUNK, W0), _f32)
    zeros64 = jnp.zeros((CHUNK, H), _f32)

    Wl0e = jnp.concatenate([Wl0, jnp.zeros((D_IN, W0 - H), _f32)], axis=1)
    bl0r = bl0.reshape(1, H)
    bl1r = bl1.reshape(1, H)
    bl2r = bl2.reshape(1, H)
    b0r = blin0.reshape(1, H)
    b1r = blin1.reshape(1, H)
    bor = bout.reshape(1, D_OUT)

    p0 = _tc1(x, Wl0e)
    agg0 = _sc_segsum_80(p0, src_p, dst_p, zeros80)
    p1, h1, rc = _tc2(agg0, x, Wr0, bl0r, Wl1)
    agg1 = _sc_segsum_64(p1, src_p, dst_p, zeros64)
    p2, h2 = _tc3(agg1, h1, Wr1, bl1r, Wl2, rc)
    agg2 = _sc_segsum_64(p2, src_p, dst_p, zeros64)
    return _tc4(agg2, h2, Wr2, bl2r, rc, Wlin0, b0r, Wlin1, b1r, Wout, bor)
